# baseline (speedup 1.0000x reference)
import functools

import jax
import jax.numpy as jnp
from jax import lax
from jax.experimental import pallas as pl
from jax.experimental.pallas import tpu as pltpu
from jax.experimental.pallas import tpu_sc as plsc

MAXTIME = 5.0
NUM_STEPS = 100
DIM = 1024
N, B = 4096, 4
L = 16
NW = 32
RPW = N // NW
CH = 8
NCHUNK = RPW // CH
NBUF = 3


def _sc_body(idx_hbm, alpha_hbm, pe_hbm, xs_hbm, out_hbm,
             idx_v, alpha_v, rows_v, pe_v, bufs_v,
             gsem, in0, in1, in2, out0, out1, out2):
    cid = lax.axis_index("c")
    sid = lax.axis_index("s")
    wid = sid * 2 + cid
    base = wid * RPW
    insems = (in0, in1, in2)
    outsems = (out0, out1, out2)

    def in_copy(g):
        k = g % NBUF
        return pltpu.make_async_copy(
            xs_hbm.at[pl.ds(base + g * CH, CH)], bufs_v.at[k], insems[k])

    def out_copy(g):
        k = g % NBUF
        return pltpu.make_async_copy(
            bufs_v.at[k], out_hbm.at[pl.ds(base + g * CH, CH)], outsems[k])

    for g in range(NBUF):
        in_copy(g).start()

    pltpu.sync_copy(idx_hbm, idx_v)
    pltpu.sync_copy(alpha_hbm, alpha_v)
    pltpu.async_copy(pe_hbm.at[idx_v], rows_v, gsem).wait()
    for b in range(B):
        ab = alpha_v[b, :]

        @plsc.parallel_loop(0, DIM // L, unroll=8)
        def _(c):
            sl = pl.ds(c * L, L)
            f = rows_v[b, sl]
            pe_v[b, sl] = f + ab * (rows_v[b + B, sl] - f)

    for g in range(NCHUNK):
        k = g % NBUF
        in_copy(g).wait()
        if g >= 2 and g + 1 < NCHUNK:
            out_copy(g - 2).wait()
            in_copy(g + 1).start()

        @plsc.parallel_loop(0, CH * B * (DIM // L), unroll=8)
        def _(j):
            r = j >> 8
            b = (j >> 6) & 3
            c = j & 63
            sl = pl.ds(c * L, L)
            bufs_v[k, r, b, sl] = bufs_v[k, r, b, sl] + pe_v[b, sl]

        out_copy(g).start()

    for g in range(NCHUNK - NBUF, NCHUNK):
        out_copy(g).wait()


def _sc_run(idx, alpha_rep, pe, xs):
    mesh = plsc.VectorSubcoreMesh(core_axis_name="c", subcore_axis_name="s")
    k = functools.partial(
        pl.kernel,
        mesh=mesh,
        out_type=jax.ShapeDtypeStruct((N, B, DIM), jnp.float32),
        scratch_types=[
            pltpu.VMEM((L,), jnp.int32),
            pltpu.VMEM((B, L), jnp.float32),
            pltpu.VMEM((L, DIM), jnp.float32),
            pltpu.VMEM((B, DIM), jnp.float32),
            pltpu.VMEM((NBUF, CH, B, DIM), jnp.float32),
            pltpu.SemaphoreType.DMA,
            pltpu.SemaphoreType.DMA,
            pltpu.SemaphoreType.DMA,
            pltpu.SemaphoreType.DMA,
            pltpu.SemaphoreType.DMA,
            pltpu.SemaphoreType.DMA,
            pltpu.SemaphoreType.DMA,
        ],
    )(_sc_body)
    return k(idx, alpha_rep, pe, xs)


def kernel(xs, times, pe):
    t = jnp.clip(times, 0.0, MAXTIME) * ((NUM_STEPS - 1) / MAXTIME)
    t_floor = jnp.floor(t)
    fi = t_floor.astype(jnp.int32)
    ci = jnp.ceil(t).astype(jnp.int32)
    alpha = t - t_floor
    idx = jnp.concatenate([fi, ci, jnp.zeros((L - 2 * B,), jnp.int32)])
    alpha_rep = jnp.broadcast_to(alpha[:, None], (B, L))
    return _sc_run(idx, alpha_rep, pe, xs)


# speedup vs baseline: 1.0045x; 1.0045x over previous
import functools

import jax
import jax.numpy as jnp
from jax import lax
from jax.experimental import pallas as pl
from jax.experimental.pallas import tpu as pltpu
from jax.experimental.pallas import tpu_sc as plsc

MAXTIME = 5.0
NUM_STEPS = 100
DIM = 1024
N, B = 4096, 4
L = 16
NW = 32
RPW = N // NW
CH = 8
NCHUNK = RPW // CH
NBUF = 3


def _sc_body(idx_hbm, alpha_hbm, pe_hbm, xs_hbm, out_hbm,
             idx_v, alpha_v, rows_v, pe_v, bufs_v,
             gsem, in0, in1, in2, out0, out1, out2):
    cid = lax.axis_index("c")
    sid = lax.axis_index("s")
    wid = sid * 2 + cid
    base = wid * RPW
    insems = (in0, in1, in2)
    outsems = (out0, out1, out2)

    def in_copy(g):
        k = g % NBUF
        return pltpu.make_async_copy(
            xs_hbm.at[pl.ds(base + g * CH, CH)], bufs_v.at[k], insems[k])

    def out_copy(g):
        k = g % NBUF
        return pltpu.make_async_copy(
            bufs_v.at[k], out_hbm.at[pl.ds(base + g * CH, CH)], outsems[k])

    for g in range(NBUF):
        in_copy(g).start()

    pltpu.sync_copy(idx_hbm, idx_v)
    pltpu.sync_copy(alpha_hbm, alpha_v)
    pltpu.async_copy(pe_hbm.at[idx_v], rows_v, gsem).wait()
    for b in range(B):
        ab = alpha_v[b, :]

        @plsc.parallel_loop(0, DIM // L, unroll=8)
        def _(c):
            sl = pl.ds(c * L, L)
            f = rows_v[b, sl]
            pe_v[b, sl] = f + ab * (rows_v[b + B, sl] - f)

    for g in range(NCHUNK):
        k = g % NBUF
        in_copy(g).wait()
        if g >= 2 and g + 1 < NCHUNK:
            out_copy(g - 2).wait()
            in_copy(g + 1).start()

        @plsc.parallel_loop(0, B * (DIM // L), unroll=2)
        def _(m):
            b = m >> 6
            c = m & 63
            sl = pl.ds(c * L, L)
            p = pe_v[b, sl]
            for r in range(CH):
                bufs_v[k, r, b, sl] = bufs_v[k, r, b, sl] + p

        out_copy(g).start()

    for g in range(NCHUNK - NBUF, NCHUNK):
        out_copy(g).wait()


def _sc_run(idx, alpha_rep, pe, xs):
    mesh = plsc.VectorSubcoreMesh(core_axis_name="c", subcore_axis_name="s")
    k = functools.partial(
        pl.kernel,
        mesh=mesh,
        out_type=jax.ShapeDtypeStruct((N, B, DIM), jnp.float32),
        scratch_types=[
            pltpu.VMEM((L,), jnp.int32),
            pltpu.VMEM((B, L), jnp.float32),
            pltpu.VMEM((L, DIM), jnp.float32),
            pltpu.VMEM((B, DIM), jnp.float32),
            pltpu.VMEM((NBUF, CH, B, DIM), jnp.float32),
            pltpu.SemaphoreType.DMA,
            pltpu.SemaphoreType.DMA,
            pltpu.SemaphoreType.DMA,
            pltpu.SemaphoreType.DMA,
            pltpu.SemaphoreType.DMA,
            pltpu.SemaphoreType.DMA,
            pltpu.SemaphoreType.DMA,
        ],
    )(_sc_body)
    return k(idx, alpha_rep, pe, xs)


def kernel(xs, times, pe):
    t = jnp.clip(times, 0.0, MAXTIME) * ((NUM_STEPS - 1) / MAXTIME)
    t_floor = jnp.floor(t)
    fi = t_floor.astype(jnp.int32)
    ci = jnp.ceil(t).astype(jnp.int32)
    alpha = t - t_floor
    idx = jnp.concatenate([fi, ci, jnp.zeros((L - 2 * B,), jnp.int32)])
    alpha_rep = jnp.broadcast_to(alpha[:, None], (B, L))
    return _sc_run(idx, alpha_rep, pe, xs)


# speedup vs baseline: 1.4380x; 1.4315x over previous
import functools

import jax
import jax.numpy as jnp
from jax import lax
from jax.experimental import pallas as pl
from jax.experimental.pallas import tpu as pltpu
from jax.experimental.pallas import tpu_sc as plsc

MAXTIME = 5.0
NUM_STEPS = 100
DIM = 1024
N, B = 4096, 4
L = 16


def _sc_interp_body(idx_hbm, alpha_hbm, pe_hbm, out_hbm,
                    idx_v, alpha_v, rows_v, out_v, sem):
    cid = lax.axis_index("c")
    sid = lax.axis_index("s")

    @pl.when(jnp.logical_and(cid == 0, sid == 0))
    def _():
        pltpu.sync_copy(idx_hbm, idx_v)
        pltpu.sync_copy(alpha_hbm, alpha_v)
        pltpu.async_copy(pe_hbm.at[idx_v], rows_v, sem).wait()
        for b in range(B):
            ab = alpha_v[b, :]
            for c in range(DIM // L):
                sl = pl.ds(c * L, L)
                f = rows_v[b, sl]
                out_v[b, sl] = f + ab * (rows_v[b + B, sl] - f)
        pltpu.sync_copy(out_v, out_hbm)


def _sc_interp(idx, alpha_rep, pe):
    mesh = plsc.VectorSubcoreMesh(core_axis_name="c", subcore_axis_name="s")
    k = functools.partial(
        pl.kernel,
        mesh=mesh,
        out_type=jax.ShapeDtypeStruct((B, DIM), jnp.float32),
        scratch_types=[
            pltpu.VMEM((L,), jnp.int32),
            pltpu.VMEM((B, L), jnp.float32),
            pltpu.VMEM((L, DIM), jnp.float32),
            pltpu.VMEM((B, DIM), jnp.float32),
            pltpu.SemaphoreType.DMA,
        ],
    )(_sc_interp_body)
    return k(idx, alpha_rep, pe)


def _tc_add_body(x_ref, p_ref, o_ref):
    o_ref[...] = x_ref[...] + p_ref[...][None]


def _tc_add(xs, pe_interp, blk):
    return pl.pallas_call(
        _tc_add_body,
        grid=(N // blk,),
        in_specs=[
            pl.BlockSpec((blk, B, DIM), lambda i: (i, 0, 0)),
            pl.BlockSpec(memory_space=pltpu.VMEM),
        ],
        out_specs=pl.BlockSpec((blk, B, DIM), lambda i: (i, 0, 0)),
        out_shape=jax.ShapeDtypeStruct((N, B, DIM), jnp.float32),
        compiler_params=pltpu.CompilerParams(vmem_limit_bytes=120 * 1024 * 1024),
    )(xs, pe_interp)


def kernel(xs, times, pe):
    t = jnp.clip(times, 0.0, MAXTIME) * ((NUM_STEPS - 1) / MAXTIME)
    t_floor = jnp.floor(t)
    fi = t_floor.astype(jnp.int32)
    ci = jnp.ceil(t).astype(jnp.int32)
    alpha = t - t_floor
    idx = jnp.concatenate([fi, ci, jnp.zeros((L - 2 * B,), jnp.int32)])
    alpha_rep = jnp.broadcast_to(alpha[:, None], (B, L))
    pe_interp = _sc_interp(idx, alpha_rep, pe)
    return _tc_add(xs, pe_interp, 512)
